# baseline (speedup 1.0000x reference)
import functools

import jax
import jax.numpy as jnp
from jax import lax
from jax.experimental import pallas as pl
from jax.experimental.pallas import tpu as pltpu
from jax.experimental.pallas import tpu_sc as plsc

BLK = 64
NB = 32
NMID = NB - 2
NSLOT = 8
SCALE = 1.0 / 8.0


def _routing_tables(attention_mask):
    bm = attention_mask[::BLK, ::BLK]
    mesh = plsc.VectorSubcoreMesh(core_axis_name="c", subcore_axis_name="s")

    @functools.partial(
        pl.kernel, mesh=mesh,
        out_type=[jax.ShapeDtypeStruct((NB, 16), jnp.int32),
                  jax.ShapeDtypeStruct((NB, 16), jnp.float32)],
        scratch_types=[pltpu.VMEM((NB,), jnp.float32),
                       pltpu.VMEM((16,), jnp.int32),
                       pltpu.VMEM((16,), jnp.float32)],
        compiler_params=pltpu.CompilerParams(needs_layout_passes=False),
    )
    def route(bm_hbm, idx_hbm, bias_hbm, row_v, idx_v, bias_v):
        wid = lax.axis_index("s") * 2 + lax.axis_index("c")
        pltpu.sync_copy(bm_hbm.at[wid], row_v)
        x0 = row_v[pl.ds(0, 16)]
        x1 = row_v[pl.ds(16, 16)]
        m0 = x0 > 0.0
        m1 = x1 > 0.0
        c0 = jnp.cumsum(m0.astype(jnp.int32))
        c1 = jnp.cumsum(m1.astype(jnp.int32)) + jnp.sum(m0.astype(jnp.int32))
        lanes = lax.broadcasted_iota(jnp.int32, (16,), 0)
        cols1 = lanes + 16
        zero = jnp.zeros((16,), jnp.int32)
        acc_idx = jnp.zeros((16,), jnp.int32)
        acc_bias = jnp.zeros((16,), jnp.float32)
        for j in range(NSLOT):
            t = j + 1
            oh0 = m0 & (c0 == t)
            oh1 = m1 & (c1 == t)
            idx_j = (jnp.sum(jnp.where(oh0, lanes, zero))
                     + jnp.sum(jnp.where(oh1, cols1, zero)))
            cnt_j = (jnp.sum(oh0.astype(jnp.int32))
                     + jnp.sum(oh1.astype(jnp.int32)))
            bias_j = jnp.where(cnt_j > 0, 0.0, -1e9)
            sel = lanes == j
            acc_idx = jnp.where(sel, idx_j, acc_idx)
            acc_bias = jnp.where(sel, bias_j, acc_bias)
        idx_v[...] = acc_idx
        bias_v[...] = acc_bias
        pltpu.sync_copy(idx_v, idx_hbm.at[wid])
        pltpu.sync_copy(bias_v, bias_hbm.at[wid])

    idx16, bias16 = route(bm)
    return idx16[1:NB - 1, :NSLOT], bias16[1:NB - 1, :NSLOT]


def _attn_kernel(idx_ref, biasrow_ref, q_ref, k_ref, v_ref, o_ref,
                 s_ref, p_ref, kb_ref, vb_ref):
    q = (q_ref[0] * SCALE).astype(jnp.bfloat16)
    kb_ref[...] = k_ref[0].astype(jnp.bfloat16)
    vb_ref[...] = v_ref[0].astype(jnp.bfloat16)
    k = kb_ref[...]
    v = vb_ref[...]

    qg = jnp.concatenate([q[:BLK], q[(NB - 1) * BLK:]], axis=0)
    s = lax.dot_general(qg, k, (((1,), (1,)), ((), ())),
                        preferred_element_type=jnp.float32)
    e = jnp.exp(s)
    p = (e / jnp.sum(e, axis=1, keepdims=True)).astype(v.dtype)
    og = lax.dot_general(p, v, (((1,), (0,)), ((), ())),
                         preferred_element_type=jnp.float32)
    o_ref[0, :BLK] = og[:BLK]
    o_ref[0, (NB - 1) * BLK:] = og[BLK:]

    for r in range(NMID):
        qr = q[(r + 1) * BLK:(r + 2) * BLK]
        kband = jnp.concatenate(
            [kb_ref[pl.ds(idx_ref[r, j] * BLK, BLK), :]
             for j in range(NSLOT)], axis=0)
        s = lax.dot_general(qr, kband, (((1,), (1,)), ((), ())),
                            preferred_element_type=jnp.float32)
        s_ref[r * BLK:(r + 1) * BLK] = s + biasrow_ref[r, 0][None, :]

    for c in range(0, NMID, 2):
        s = s_ref[c * BLK:(c + 2) * BLK]
        e = jnp.exp(s)
        p_ref[c * BLK:(c + 2) * BLK] = (
            e / jnp.sum(e, axis=1, keepdims=True)).astype(jnp.bfloat16)

    for r in range(NMID):
        vband = jnp.concatenate(
            [vb_ref[pl.ds(idx_ref[r, j] * BLK, BLK), :]
             for j in range(NSLOT)], axis=0)
        acc = lax.dot_general(p_ref[r * BLK:(r + 1) * BLK], vband,
                              (((1,), (0,)), ((), ())),
                              preferred_element_type=jnp.float32)
        o_ref[0, (r + 1) * BLK:(r + 2) * BLK] = acc


@jax.jit
def kernel(query_layer, key_layer, value_layer, attention_mask):
    b, h, sq, d = query_layer.shape
    bh = b * h
    sk = key_layer.shape[2]
    q3 = query_layer.reshape(bh, sq, d)
    k3 = key_layer.reshape(bh, sk, d)
    v3 = value_layer.reshape(bh, sk, d)
    idx, bias = _routing_tables(attention_mask)
    biasrow = jnp.repeat(bias, BLK, axis=1).reshape(NMID, 1, NSLOT * BLK)

    grid = (bh,)
    bf_spec = pl.BlockSpec((1, sq, d), lambda i: (i, 0, 0))
    smem_spec = pl.BlockSpec(memory_space=pltpu.SMEM)
    biasrow_spec = pl.BlockSpec((NMID, 1, NSLOT * BLK), lambda i: (0, 0, 0))
    out = pl.pallas_call(
        _attn_kernel,
        grid=grid,
        in_specs=[smem_spec, biasrow_spec, bf_spec, bf_spec, bf_spec],
        out_specs=bf_spec,
        out_shape=jax.ShapeDtypeStruct((bh, sq, d), jnp.float32),
        scratch_shapes=[
            pltpu.VMEM((NMID * BLK, NSLOT * BLK), jnp.float32),
            pltpu.VMEM((NMID * BLK, NSLOT * BLK), jnp.bfloat16),
            pltpu.VMEM((sk, d), jnp.bfloat16),
            pltpu.VMEM((sk, d), jnp.bfloat16),
        ],
    )(idx, biasrow, q3, k3, v3)
    return out.reshape(b, h, sq, d)
